# baseline (device time: 19343 ns/iter reference)
import jax
import jax.numpy as jnp
from jax import lax
from jax.experimental import pallas as pl
from jax.experimental.pallas import tpu as pltpu

N_DEV = 8
B = 2
SQ = 128
SKV = 1024
HQ = 4
DH = 64
D_MODEL = 512
D_QK = HQ * DH
KV_PER = SKV // N_DEV
RQ = SQ // N_DEV
NEG = -1e9


def kernel(x, Wq, K_ext, V_ext, Wo):
    k_shard = K_ext.reshape(B, KV_PER, D_QK)
    v_shard = V_ext.reshape(B, KV_PER, D_QK)

    def body(x_ref, wq_ref, k_ref, v_ref, wo_ref, out_ref,
             upart, spart, ucomb, scomb, omine,
             usend, urecv, ssend, srecv, osend, orecv):
        my = lax.axis_index("i")

        barrier_sem = pltpu.get_barrier_semaphore()
        for nbr in range(N_DEV):
            @pl.when(nbr != my)
            def _():
                pl.semaphore_signal(
                    barrier_sem, inc=1,
                    device_id=(nbr,), device_id_type=pl.DeviceIdType.MESH,
                )
        pl.semaphore_wait(barrier_sem, N_DEV - 1)

        qi = lax.broadcasted_iota(jnp.int32, (SQ, KV_PER), 0)
        kj = lax.broadcasted_iota(jnp.int32, (SQ, KV_PER), 1) + my * KV_PER
        mask = (jnp.abs(qi - kj) <= 128) | (kj < 32) | (qi < 32)

        for b in range(B):
            q_b = jnp.dot(x_ref[b], wq_ref[...],
                          preferred_element_type=jnp.float32)
            ms, ls = [], []
            for hh in range(HQ):
                sl = slice(hh * DH, (hh + 1) * DH)
                scores = lax.dot_general(
                    q_b[:, sl], k_ref[b][:, sl],
                    (((1,), (1,)), ((), ())),
                    preferred_element_type=jnp.float32,
                )
                s = jnp.where(mask, scores * 0.125, NEG)
                m = jnp.max(s, axis=1, keepdims=True)
                p = jnp.exp(s - m)
                l = jnp.sum(p, axis=1, keepdims=True)
                u = jnp.dot(p, v_ref[b][:, sl],
                            preferred_element_type=jnp.float32)
                ub = u.astype(jnp.bfloat16)
                for d in range(N_DEV):
                    upart[d, b, :, sl] = ub[d * RQ:(d + 1) * RQ]
                ms.append(m)
                ls.append(l)
            st = jnp.concatenate(ms + ls, axis=1)
            for d in range(N_DEV):
                spart[d, b] = st[d * RQ:(d + 1) * RQ]

            for peer in range(N_DEV):
                @pl.when(peer != my)
                def _():
                    pltpu.make_async_remote_copy(
                        src_ref=upart.at[peer, b], dst_ref=ucomb.at[my, b],
                        send_sem=usend.at[peer, b], recv_sem=urecv.at[my, b],
                        device_id=(peer,), device_id_type=pl.DeviceIdType.MESH,
                    ).start()
                    pltpu.make_async_remote_copy(
                        src_ref=spart.at[peer, b], dst_ref=scomb.at[my, b],
                        send_sem=ssend.at[peer, b], recv_sem=srecv.at[my, b],
                        device_id=(peer,), device_id_type=pl.DeviceIdType.MESH,
                    ).start()
            ucomb[pl.ds(my, 1), b] = upart[my, b][None]
            scomb[pl.ds(my, 1), b] = spart[my, b][None]

        for b in range(B):
            for o in range(N_DEV):
                @pl.when(o != my)
                def _():
                    pltpu.make_async_remote_copy(
                        src_ref=ucomb.at[o, b], dst_ref=ucomb.at[o, b],
                        send_sem=usend.at[o, b], recv_sem=urecv.at[o, b],
                        device_id=(my,), device_id_type=pl.DeviceIdType.MESH,
                    ).wait_recv()
                    pltpu.make_async_remote_copy(
                        src_ref=scomb.at[o, b], dst_ref=scomb.at[o, b],
                        send_sem=ssend.at[o, b], recv_sem=srecv.at[o, b],
                        device_id=(my,), device_id_type=pl.DeviceIdType.MESH,
                    ).wait_recv()

            ctx_heads = []
            for hh in range(HQ):
                sl = slice(hh * DH, (hh + 1) * DH)
                ms = [scomb[o, b][:, hh:hh + 1] for o in range(N_DEV)]
                ls = [scomb[o, b][:, HQ + hh:HQ + hh + 1] for o in range(N_DEV)]
                M = ms[0]
                for o in range(1, N_DEV):
                    M = jnp.maximum(M, ms[o])
                ws = [jnp.exp(ms[o] - M) for o in range(N_DEV)]
                L = ws[0] * ls[0]
                acc = ws[0] * ucomb[0, b][:, sl].astype(jnp.float32)
                for o in range(1, N_DEV):
                    L = L + ws[o] * ls[o]
                    acc = acc + ws[o] * ucomb[o, b][:, sl].astype(jnp.float32)
                ctx_heads.append(acc / L)
            ctx = jnp.concatenate(ctx_heads, axis=1)
            o_b = jnp.dot(ctx, wo_ref[...],
                          preferred_element_type=jnp.float32)
            omine[b] = o_b
            out_ref[b, pl.ds(my * RQ, RQ), :] = o_b

            for peer in range(N_DEV):
                @pl.when(peer != my)
                def _():
                    pltpu.make_async_remote_copy(
                        src_ref=omine.at[b],
                        dst_ref=out_ref.at[b, pl.ds(my * RQ, RQ)],
                        send_sem=osend.at[peer, b], recv_sem=orecv.at[my, b],
                        device_id=(peer,), device_id_type=pl.DeviceIdType.MESH,
                    ).start()

        for b in range(B):
            for o in range(N_DEV):
                @pl.when(o != my)
                def _():
                    pltpu.make_async_remote_copy(
                        src_ref=omine.at[b],
                        dst_ref=out_ref.at[b, pl.ds(o * RQ, RQ)],
                        send_sem=osend.at[o, b], recv_sem=orecv.at[o, b],
                        device_id=(my,), device_id_type=pl.DeviceIdType.MESH,
                    ).wait_recv()

        for b in range(B):
            for peer in range(N_DEV):
                @pl.when(peer != my)
                def _():
                    pltpu.make_async_remote_copy(
                        src_ref=upart.at[peer, b], dst_ref=ucomb.at[my, b],
                        send_sem=usend.at[peer, b], recv_sem=urecv.at[my, b],
                        device_id=(peer,), device_id_type=pl.DeviceIdType.MESH,
                    ).wait_send()
                    pltpu.make_async_remote_copy(
                        src_ref=spart.at[peer, b], dst_ref=scomb.at[my, b],
                        send_sem=ssend.at[peer, b], recv_sem=srecv.at[my, b],
                        device_id=(peer,), device_id_type=pl.DeviceIdType.MESH,
                    ).wait_send()
                    pltpu.make_async_remote_copy(
                        src_ref=omine.at[b],
                        dst_ref=out_ref.at[b, pl.ds(my * RQ, RQ)],
                        send_sem=osend.at[peer, b], recv_sem=orecv.at[my, b],
                        device_id=(peer,), device_id_type=pl.DeviceIdType.MESH,
                    ).wait_send()

    return pl.pallas_call(
        body,
        out_shape=jax.ShapeDtypeStruct((B, SQ, D_MODEL), jnp.float32),
        in_specs=[pl.BlockSpec(memory_space=pltpu.VMEM)] * 5,
        out_specs=pl.BlockSpec(memory_space=pltpu.VMEM),
        scratch_shapes=[
            pltpu.VMEM((N_DEV, B, RQ, D_QK), jnp.bfloat16),
            pltpu.VMEM((N_DEV, B, RQ, 2 * HQ), jnp.float32),
            pltpu.VMEM((N_DEV, B, RQ, D_QK), jnp.bfloat16),
            pltpu.VMEM((N_DEV, B, RQ, 2 * HQ), jnp.float32),
            pltpu.VMEM((B, RQ, D_MODEL), jnp.float32),
            pltpu.SemaphoreType.DMA((N_DEV, B)),
            pltpu.SemaphoreType.DMA((N_DEV, B)),
            pltpu.SemaphoreType.DMA((N_DEV, B)),
            pltpu.SemaphoreType.DMA((N_DEV, B)),
            pltpu.SemaphoreType.DMA((N_DEV, B)),
            pltpu.SemaphoreType.DMA((N_DEV, B)),
        ],
        compiler_params=pltpu.CompilerParams(collective_id=0),
    )(x, Wq, k_shard, v_shard, Wo)


# device time: 18889 ns/iter; 1.0240x vs baseline; 1.0240x over previous
import jax
import jax.numpy as jnp
from jax import lax
from jax.experimental import pallas as pl
from jax.experimental.pallas import tpu as pltpu

N_DEV = 8
B = 2
SQ = 128
SKV = 1024
HQ = 4
DH = 64
D_MODEL = 512
D_QK = HQ * DH
KV_PER = SKV // N_DEV
RQ = SQ // N_DEV
NEG = -1e9


def kernel(x, Wq, K_ext, V_ext, Wo):
    k_shard = K_ext.reshape(B, KV_PER, D_QK)
    v_shard = V_ext.reshape(B, KV_PER, D_QK)

    def body(x_ref, wq_ref, k_ref, v_ref, wo_ref, out_ref,
             upart, spart, ucomb, scomb, omine, oall,
             usend, urecv, ssend, srecv, osend, orecv):
        my = lax.axis_index("i")

        barrier_sem = pltpu.get_barrier_semaphore()
        for nbr in range(N_DEV):
            @pl.when(nbr != my)
            def _():
                pl.semaphore_signal(
                    barrier_sem, inc=1,
                    device_id=(nbr,), device_id_type=pl.DeviceIdType.MESH,
                )
        pl.semaphore_wait(barrier_sem, N_DEV - 1)

        qi = lax.broadcasted_iota(jnp.int32, (SQ, KV_PER), 0)
        kj = lax.broadcasted_iota(jnp.int32, (SQ, KV_PER), 1) + my * KV_PER
        mask = (jnp.abs(qi - kj) <= 128) | (kj < 32) | (qi < 32)

        for b in range(B):
            q_b = jnp.dot(x_ref[b], wq_ref[...],
                          preferred_element_type=jnp.float32)
            ms, ls = [], []
            for hh in range(HQ):
                sl = slice(hh * DH, (hh + 1) * DH)
                scores = lax.dot_general(
                    q_b[:, sl], k_ref[b][:, sl],
                    (((1,), (1,)), ((), ())),
                    preferred_element_type=jnp.float32,
                )
                s = jnp.where(mask, scores * 0.125, NEG)
                m = jnp.max(s, axis=1, keepdims=True)
                p = jnp.exp(s - m)
                l = jnp.sum(p, axis=1, keepdims=True)
                u = jnp.dot(p, v_ref[b][:, sl],
                            preferred_element_type=jnp.float32)
                ub = u.astype(jnp.bfloat16)
                for d in range(N_DEV):
                    upart[d, b, :, sl] = ub[d * RQ:(d + 1) * RQ]
                ms.append(m)
                ls.append(l)
            st = jnp.concatenate(ms + ls, axis=1)
            for d in range(N_DEV):
                spart[d, b] = st[d * RQ:(d + 1) * RQ]

            for peer in range(N_DEV):
                @pl.when(peer != my)
                def _():
                    pltpu.make_async_remote_copy(
                        src_ref=upart.at[peer, b], dst_ref=ucomb.at[my, b],
                        send_sem=usend.at[peer, b], recv_sem=urecv.at[my, b],
                        device_id=(peer,), device_id_type=pl.DeviceIdType.MESH,
                    ).start()
                    pltpu.make_async_remote_copy(
                        src_ref=spart.at[peer, b], dst_ref=scomb.at[my, b],
                        send_sem=ssend.at[peer, b], recv_sem=srecv.at[my, b],
                        device_id=(peer,), device_id_type=pl.DeviceIdType.MESH,
                    ).start()
            ucomb[pl.ds(my, 1), b] = upart[my, b][None]
            scomb[pl.ds(my, 1), b] = spart[my, b][None]

        for b in range(B):
            for o in range(N_DEV):
                @pl.when(o != my)
                def _():
                    pltpu.make_async_remote_copy(
                        src_ref=ucomb.at[o, b], dst_ref=ucomb.at[o, b],
                        send_sem=usend.at[o, b], recv_sem=urecv.at[o, b],
                        device_id=(my,), device_id_type=pl.DeviceIdType.MESH,
                    ).wait_recv()
                    pltpu.make_async_remote_copy(
                        src_ref=scomb.at[o, b], dst_ref=scomb.at[o, b],
                        send_sem=ssend.at[o, b], recv_sem=srecv.at[o, b],
                        device_id=(my,), device_id_type=pl.DeviceIdType.MESH,
                    ).wait_recv()

            ctx_heads = []
            for hh in range(HQ):
                sl = slice(hh * DH, (hh + 1) * DH)
                ms = [scomb[o, b][:, hh:hh + 1] for o in range(N_DEV)]
                ls = [scomb[o, b][:, HQ + hh:HQ + hh + 1] for o in range(N_DEV)]
                M = ms[0]
                for o in range(1, N_DEV):
                    M = jnp.maximum(M, ms[o])
                ws = [jnp.exp(ms[o] - M) for o in range(N_DEV)]
                L = ws[0] * ls[0]
                acc = ws[0] * ucomb[0, b][:, sl].astype(jnp.float32)
                for o in range(1, N_DEV):
                    L = L + ws[o] * ls[o]
                    acc = acc + ws[o] * ucomb[o, b][:, sl].astype(jnp.float32)
                ctx_heads.append(acc / L)
            ctx = jnp.concatenate(ctx_heads, axis=1)
            o_b = jnp.dot(ctx, wo_ref[...],
                          preferred_element_type=jnp.float32)
            omine[b] = o_b.astype(jnp.bfloat16)
            oall[pl.ds(my, 1), b] = omine[b][None]

            for peer in range(N_DEV):
                @pl.when(peer != my)
                def _():
                    pltpu.make_async_remote_copy(
                        src_ref=omine.at[b],
                        dst_ref=oall.at[my, b],
                        send_sem=osend.at[peer, b], recv_sem=orecv.at[my, b],
                        device_id=(peer,), device_id_type=pl.DeviceIdType.MESH,
                    ).start()

        for b in range(B):
            for o in range(N_DEV):
                @pl.when(o != my)
                def _():
                    pltpu.make_async_remote_copy(
                        src_ref=omine.at[b],
                        dst_ref=oall.at[o, b],
                        send_sem=osend.at[o, b], recv_sem=orecv.at[o, b],
                        device_id=(my,), device_id_type=pl.DeviceIdType.MESH,
                    ).wait_recv()
            for o in range(N_DEV):
                out_ref[b, o * RQ:(o + 1) * RQ, :] = (
                    oall[o, b].astype(jnp.float32))

        for b in range(B):
            for peer in range(N_DEV):
                @pl.when(peer != my)
                def _():
                    pltpu.make_async_remote_copy(
                        src_ref=upart.at[peer, b], dst_ref=ucomb.at[my, b],
                        send_sem=usend.at[peer, b], recv_sem=urecv.at[my, b],
                        device_id=(peer,), device_id_type=pl.DeviceIdType.MESH,
                    ).wait_send()
                    pltpu.make_async_remote_copy(
                        src_ref=spart.at[peer, b], dst_ref=scomb.at[my, b],
                        send_sem=ssend.at[peer, b], recv_sem=srecv.at[my, b],
                        device_id=(peer,), device_id_type=pl.DeviceIdType.MESH,
                    ).wait_send()
                    pltpu.make_async_remote_copy(
                        src_ref=omine.at[b],
                        dst_ref=oall.at[my, b],
                        send_sem=osend.at[peer, b], recv_sem=orecv.at[my, b],
                        device_id=(peer,), device_id_type=pl.DeviceIdType.MESH,
                    ).wait_send()

    return pl.pallas_call(
        body,
        out_shape=jax.ShapeDtypeStruct((B, SQ, D_MODEL), jnp.float32),
        in_specs=[pl.BlockSpec(memory_space=pltpu.VMEM)] * 5,
        out_specs=pl.BlockSpec(memory_space=pltpu.VMEM),
        scratch_shapes=[
            pltpu.VMEM((N_DEV, B, RQ, D_QK), jnp.bfloat16),
            pltpu.VMEM((N_DEV, B, RQ, 2 * HQ), jnp.float32),
            pltpu.VMEM((N_DEV, B, RQ, D_QK), jnp.bfloat16),
            pltpu.VMEM((N_DEV, B, RQ, 2 * HQ), jnp.float32),
            pltpu.VMEM((B, RQ, D_MODEL), jnp.bfloat16),
            pltpu.VMEM((N_DEV, B, RQ, D_MODEL), jnp.bfloat16),
            pltpu.SemaphoreType.DMA((N_DEV, B)),
            pltpu.SemaphoreType.DMA((N_DEV, B)),
            pltpu.SemaphoreType.DMA((N_DEV, B)),
            pltpu.SemaphoreType.DMA((N_DEV, B)),
            pltpu.SemaphoreType.DMA((N_DEV, B)),
            pltpu.SemaphoreType.DMA((N_DEV, B)),
        ],
        compiler_params=pltpu.CompilerParams(collective_id=0),
    )(x, Wq, k_shard, v_shard, Wo)
